# baseline (speedup 1.0000x reference)
import functools
import math

import jax
import jax.numpy as jnp
from jax import lax
from jax.experimental import pallas as pl
from jax.experimental.pallas import tpu as pltpu
from jax.experimental.pallas import tpu_sc as plsc

N_ITEMS = 1000000
D_DIM = 32
S_SAMPLES = 1024

BLOCK_ITEMS = 167936
NUM_BLOCKS = -(-N_ITEMS // BLOCK_ITEMS)

NUM_WORKERS = 32
IDX_PER_WORKER = S_SAMPLES // NUM_WORKERS
LANES = 16

_LOG_EPS = math.log(1e-8)
_NEG_BIG = -1e30


def _tc_score_body(et_ref, b_ref, ut_ref, s_ref, m_ref, z_ref):
    i = pl.program_id(0)
    s = lax.dot_general(ut_ref[...], et_ref[...], (((1,), (0,)), ((), ())),
                        preferred_element_type=jnp.float32
                        ).reshape((BLOCK_ITEMS,)) + b_ref[...]
    s_ref[...] = s
    pos = i * BLOCK_ITEMS + lax.iota(jnp.int32, BLOCK_ITEMS)
    sm = jnp.where(pos < N_ITEMS, s, _NEG_BIG)
    m_blk = jnp.max(sm)
    z_blk = jnp.sum(jnp.exp(sm - m_blk))
    m_ref[...] = jnp.full((128,), m_blk)
    z_ref[...] = jnp.full((128,), z_blk)


def _tc_scores(Et, B, ut):
    return pl.pallas_call(
        _tc_score_body,
        grid=(NUM_BLOCKS,),
        in_specs=[
            pl.BlockSpec((D_DIM, BLOCK_ITEMS), lambda i: (0, i)),
            pl.BlockSpec((BLOCK_ITEMS,), lambda i: (i,)),
            pl.BlockSpec((1, D_DIM), lambda i: (0, 0)),
        ],
        out_specs=[
            pl.BlockSpec((BLOCK_ITEMS,), lambda i: (i,)),
            pl.BlockSpec((128,), lambda i: (i,)),
            pl.BlockSpec((128,), lambda i: (i,)),
        ],
        out_shape=[
            jax.ShapeDtypeStruct((N_ITEMS,), jnp.float32),
            jax.ShapeDtypeStruct((NUM_BLOCKS * 128,), jnp.float32),
            jax.ShapeDtypeStruct((NUM_BLOCKS * 128,), jnp.float32),
        ],
        compiler_params=pltpu.CompilerParams(
            dimension_semantics=("parallel",),
        ),
    )(Et, B, ut)


def _sc_gather(scores, idx):
    mesh = plsc.VectorSubcoreMesh(core_axis_name="c", subcore_axis_name="s")

    @functools.partial(
        pl.kernel,
        mesh=mesh,
        out_type=jax.ShapeDtypeStruct((S_SAMPLES,), jnp.float32),
        scratch_types=[
            pltpu.VMEM((IDX_PER_WORKER,), jnp.int32),
            pltpu.VMEM((IDX_PER_WORKER,), jnp.float32),
            pltpu.SemaphoreType.DMA,
        ],
    )
    def gather_kernel(s_hbm, idx_hbm, out_hbm, idx_v, g_v, sem):
        wid = lax.axis_index("s") * 2 + lax.axis_index("c")
        base = wid * IDX_PER_WORKER
        pltpu.sync_copy(idx_hbm.at[pl.ds(base, IDX_PER_WORKER)], idx_v)
        cp = pltpu.async_copy(s_hbm.at[idx_v], g_v, sem)
        cp.wait()
        pltpu.sync_copy(g_v, out_hbm.at[pl.ds(base, IDX_PER_WORKER)])

    return gather_kernel(scores, idx)


def _tc_loss_body(s_ref, rew_ref, m_ref, z_ref, out_ref):
    m_vec = m_ref[...]
    z_vec = z_ref[...]
    m_all = jnp.max(m_vec)
    z_all = jnp.sum(z_vec * jnp.exp(m_vec - m_all)) * (1.0 / 128.0)
    c = m_all + jnp.log(z_all)
    logp = jnp.maximum(s_ref[...] - c, _LOG_EPS)
    out_ref[...] = jnp.full((1, 1), -jnp.mean(logp * rew_ref[...]))


def kernel(G_user_embeddings, G_item_embeddings, G_item_bias, user_index,
           sample, reward):
    ut = lax.dynamic_slice_in_dim(G_user_embeddings, user_index, 1, axis=0)
    idx = sample.astype(jnp.int32)

    Et = jnp.transpose(G_item_embeddings)

    s, m_vec, z_vec = _tc_scores(Et, G_item_bias, ut)
    s_smp = _sc_gather(s, idx)

    loss = pl.pallas_call(
        _tc_loss_body,
        out_shape=jax.ShapeDtypeStruct((1, 1), jnp.float32),
    )(s_smp, reward, m_vec, z_vec)
    return loss.reshape(())


# speedup vs baseline: 1.0339x; 1.0339x over previous
import functools
import math

import jax
import jax.numpy as jnp
from jax import lax
from jax.experimental import pallas as pl
from jax.experimental.pallas import tpu as pltpu
from jax.experimental.pallas import tpu_sc as plsc

N_ITEMS = 1000000
D_DIM = 32
S_SAMPLES = 1024

BLOCK_ITEMS = 102400
NUM_BLOCKS = -(-N_ITEMS // BLOCK_ITEMS)

NUM_WORKERS = 32
IDX_PER_WORKER = S_SAMPLES // NUM_WORKERS
LANES = 16

_LOG_EPS = math.log(1e-8)
_NEG_BIG = -1e30


def _tc_score_body(et_ref, b_ref, ut_ref, s_ref, m_ref, z_ref):
    i = pl.program_id(0)
    s = lax.dot_general(ut_ref[...], et_ref[...], (((1,), (0,)), ((), ())),
                        preferred_element_type=jnp.float32
                        ).reshape((BLOCK_ITEMS,)) + b_ref[...]
    s_ref[...] = s
    pos = i * BLOCK_ITEMS + lax.iota(jnp.int32, BLOCK_ITEMS)
    sm = jnp.where(pos < N_ITEMS, s, _NEG_BIG)
    m_blk = jnp.max(sm)
    z_blk = jnp.sum(jnp.exp(sm - m_blk))
    m_ref[...] = jnp.full((128,), m_blk)
    z_ref[...] = jnp.full((128,), z_blk)


def _tc_scores(Et, B, ut):
    return pl.pallas_call(
        _tc_score_body,
        grid=(NUM_BLOCKS,),
        in_specs=[
            pl.BlockSpec((D_DIM, BLOCK_ITEMS), lambda i: (0, i)),
            pl.BlockSpec((BLOCK_ITEMS,), lambda i: (i,)),
            pl.BlockSpec((1, D_DIM), lambda i: (0, 0)),
        ],
        out_specs=[
            pl.BlockSpec((BLOCK_ITEMS,), lambda i: (i,)),
            pl.BlockSpec((128,), lambda i: (i,)),
            pl.BlockSpec((128,), lambda i: (i,)),
        ],
        out_shape=[
            jax.ShapeDtypeStruct((N_ITEMS,), jnp.float32),
            jax.ShapeDtypeStruct((NUM_BLOCKS * 128,), jnp.float32),
            jax.ShapeDtypeStruct((NUM_BLOCKS * 128,), jnp.float32),
        ],
        compiler_params=pltpu.CompilerParams(
            dimension_semantics=("parallel",),
        ),
    )(Et, B, ut)


def _sc_gather(scores, idx):
    mesh = plsc.VectorSubcoreMesh(core_axis_name="c", subcore_axis_name="s")

    @functools.partial(
        pl.kernel,
        mesh=mesh,
        out_type=jax.ShapeDtypeStruct((S_SAMPLES,), jnp.float32),
        scratch_types=[
            pltpu.VMEM((IDX_PER_WORKER,), jnp.int32),
            pltpu.VMEM((IDX_PER_WORKER,), jnp.float32),
            pltpu.SemaphoreType.DMA,
        ],
    )
    def gather_kernel(s_hbm, idx_hbm, out_hbm, idx_v, g_v, sem):
        wid = lax.axis_index("s") * 2 + lax.axis_index("c")
        base = wid * IDX_PER_WORKER
        pltpu.sync_copy(idx_hbm.at[pl.ds(base, IDX_PER_WORKER)], idx_v)
        cp = pltpu.async_copy(s_hbm.at[idx_v], g_v, sem)
        cp.wait()
        pltpu.sync_copy(g_v, out_hbm.at[pl.ds(base, IDX_PER_WORKER)])

    return gather_kernel(scores, idx)


def _tc_loss_body(s_ref, rew_ref, m_ref, z_ref, out_ref):
    m_vec = m_ref[...]
    z_vec = z_ref[...]
    m_all = jnp.max(m_vec)
    z_all = jnp.sum(z_vec * jnp.exp(m_vec - m_all)) * (1.0 / 128.0)
    c = m_all + jnp.log(z_all)
    logp = jnp.maximum(s_ref[...] - c, _LOG_EPS)
    out_ref[...] = jnp.full((1, 1), -jnp.mean(logp * rew_ref[...]))


def kernel(G_user_embeddings, G_item_embeddings, G_item_bias, user_index,
           sample, reward):
    ut = lax.dynamic_slice_in_dim(G_user_embeddings, user_index, 1, axis=0)
    idx = sample.astype(jnp.int32)

    Et = jnp.transpose(G_item_embeddings)

    s, m_vec, z_vec = _tc_scores(Et, G_item_bias, ut)
    s_smp = _sc_gather(s, idx)

    loss = pl.pallas_call(
        _tc_loss_body,
        out_shape=jax.ShapeDtypeStruct((1, 1), jnp.float32),
    )(s_smp, reward, m_vec, z_vec)
    return loss.reshape(())
